# baseline (speedup 1.0000x reference)
import functools

import jax
import jax.numpy as jnp
from jax import lax
from jax.experimental import pallas as pl
from jax.experimental.pallas import tpu as pltpu
from jax.experimental.pallas import tpu_sc as plsc

N = 100000
D = 64
E = 1024
NEG = 4

NUM_CORES = 2
NUM_SUBCORES = 16
NW = NUM_CORES * NUM_SUBCORES

W = 2048
NB = 49
MPAD = W * NB
MSLICE = MPAD // NUM_SUBCORES

UH_T = E // NUM_SUBCORES
IB_T = E // NUM_SUBCORES
IT_T = E * NEG // NUM_SUBCORES

GP_T = E // NW
GH_T = E * NEG // NW
GT_T = E // NW
KFIRE = 16


def _sc_counts_body(uh_hbm, ib_hbm, it_hbm, z_hbm,
                    mh_out, mb_out, mt_out,
                    idx_s, idx_l, ones_v, vbuf,
                    acc_h, acc_b, acc_t, sem):
    cid = lax.axis_index("c")
    sid = lax.axis_index("s")
    for j in range(IT_T // 16):
        ones_v[pl.ds(j * 16, 16)] = jnp.full((16,), 1.0, jnp.float32)

    pltpu.sync_copy(z_hbm.at[pl.ds(sid * MSLICE, MSLICE)], vbuf)

    @pl.when(cid == 0)
    def _():
        pltpu.sync_copy(vbuf, acc_h.at[pl.ds(sid * MSLICE, MSLICE)])
        pltpu.sync_copy(vbuf, acc_b.at[pl.ds(sid * MSLICE, MSLICE)])

    @pl.when(cid == 1)
    def _():
        pltpu.sync_copy(vbuf, acc_t.at[pl.ds(sid * MSLICE, MSLICE)])

    plsc.subcore_barrier()

    @pl.when(cid == 0)
    def _():
        pltpu.sync_copy(uh_hbm.at[pl.ds(sid * UH_T, UH_T)], idx_s)
        pltpu.sync_copy(ones_v.at[pl.ds(0, UH_T)], acc_h.at[idx_s], add=True)
        pltpu.sync_copy(ib_hbm.at[pl.ds(sid * IB_T, IB_T)], idx_s)
        pltpu.sync_copy(ones_v.at[pl.ds(0, IB_T)], acc_b.at[idx_s], add=True)

    @pl.when(cid == 1)
    def _():
        pltpu.sync_copy(it_hbm.at[pl.ds(sid * IT_T, IT_T)], idx_l)
        pltpu.sync_copy(ones_v, acc_t.at[idx_l], add=True)

    plsc.subcore_barrier()

    @pl.when(cid == 0)
    def _():
        pltpu.sync_copy(acc_h.at[pl.ds(sid * MSLICE, MSLICE)], vbuf)
        pltpu.sync_copy(vbuf, mh_out.at[pl.ds(sid * MSLICE, MSLICE)])
        pltpu.sync_copy(acc_b.at[pl.ds(sid * MSLICE, MSLICE)], vbuf)
        pltpu.sync_copy(vbuf, mb_out.at[pl.ds(sid * MSLICE, MSLICE)])

    @pl.when(cid == 1)
    def _():
        pltpu.sync_copy(acc_t.at[pl.ds(sid * MSLICE, MSLICE)], vbuf)
        pltpu.sync_copy(vbuf, mt_out.at[pl.ds(sid * MSLICE, MSLICE)])


def _run_sc_counts(uh, ib, it, zeros_m):
    mesh = plsc.VectorSubcoreMesh(core_axis_name="c", subcore_axis_name="s")
    f = functools.partial(
        pl.kernel,
        mesh=mesh,
        out_type=[jax.ShapeDtypeStruct((MPAD,), jnp.float32)] * 3,
        scratch_types=[
            pltpu.VMEM((IB_T,), jnp.int32),
            pltpu.VMEM((IT_T,), jnp.int32),
            pltpu.VMEM((IT_T,), jnp.float32),
            pltpu.VMEM((MSLICE,), jnp.float32),
            pltpu.VMEM_SHARED((MPAD,), jnp.float32),
            pltpu.VMEM_SHARED((MPAD,), jnp.float32),
            pltpu.VMEM_SHARED((MPAD,), jnp.float32),
            pltpu.SemaphoreType.DMA,
        ],
    )(_sc_counts_body)
    return f(uh, ib, it, zeros_m)


def _tc_stream_body(tu_ref, ti_ref, k_ref, mh_ref, mb_ref, mt_ref,
                    yp_ref, yh_ref, yt_ref, s_ref, u_ref):
    g = pl.program_id(0)
    blk = lax.rem(g, NB)
    ds1 = (((1,), (1,)), ((), ()))
    cm = (((1,), (0,)), ((), ()))
    c0 = (((0,), (0,)), ((), ()))

    @pl.when(g < NB)
    def _phase_a():
        col = blk * W + lax.broadcasted_iota(jnp.int32, (1, W), 1)
        valid = col < N
        tu = jnp.where(valid, tu_ref[...], 0.0)
        ti = jnp.where(valid, ti_ref[...], 0.0)
        ph = lax.dot_general(tu, mh_ref[...], ds1,
                             preferred_element_type=jnp.float32)
        pb = lax.dot_general(ti, mb_ref[...], ds1,
                             preferred_element_type=jnp.float32)
        pt = lax.dot_general(ti, mt_ref[...], ds1,
                             preferred_element_type=jnp.float32)

        @pl.when(g == 0)
        def _():
            s_ref[:, 0:1] = ph
            s_ref[:, 1:2] = pb
            s_ref[:, 2:3] = pt

        @pl.when(g > 0)
        def _():
            s_ref[:, 0:1] = s_ref[:, 0:1] + ph
            s_ref[:, 1:2] = s_ref[:, 1:2] + pb
            s_ref[:, 2:3] = s_ref[:, 2:3] + pt

        zp = jnp.zeros((W, 8), jnp.float32)
        yp_ref[...] = zp
        yh_ref[...] = zp
        yt_ref[...] = zp

    @pl.when(g >= NB)
    def _phase_b():
        rel = k_ref[...]

        @pl.when(g == NB)
        def _():
            u_ref[:, 0:1] = lax.dot_general(
                rel, s_ref[:, 1:2], cm, preferred_element_type=jnp.float32)
            u_ref[:, 1:2] = float(NEG) * lax.dot_general(
                rel, s_ref[:, 0:1], cm, preferred_element_type=jnp.float32)
            u_ref[:, 2:3] = lax.dot_general(
                rel, s_ref[:, 2:3], cm, preferred_element_type=jnp.float32)

        tu = tu_ref[...]
        ti = ti_ref[...]
        ypc = lax.dot_general(tu, u_ref[:, 0:1], c0,
                              preferred_element_type=jnp.float32)
        yhc = lax.dot_general(tu, u_ref[:, 1:2], c0,
                              preferred_element_type=jnp.float32)
        ytc = lax.dot_general(ti, u_ref[:, 2:3], c0,
                              preferred_element_type=jnp.float32)
        yp_ref[...] = jnp.broadcast_to(ypc, (W, 8))
        yh_ref[...] = jnp.broadcast_to(yhc, (W, 8))
        yt_ref[...] = jnp.broadcast_to(ytc, (W, 8))


def _run_tc_stream(tu, ti, rel, mh, mb, mt):
    node_blk = lambda g: (0, lax.rem(g, NB))
    return pl.pallas_call(
        _tc_stream_body,
        grid=(2 * NB,),
        in_specs=[
            pl.BlockSpec((D, W), node_blk),
            pl.BlockSpec((D, W), node_blk),
            pl.BlockSpec((D, D), lambda g: (0, 0)),
            pl.BlockSpec((1, W), node_blk),
            pl.BlockSpec((1, W), node_blk),
            pl.BlockSpec((1, W), node_blk),
        ],
        out_specs=[
            pl.BlockSpec((W, 8), lambda g: (lax.rem(g, NB), 0)),
            pl.BlockSpec((W, 8), lambda g: (lax.rem(g, NB), 0)),
            pl.BlockSpec((W, 8), lambda g: (lax.rem(g, NB), 0)),
        ],
        out_shape=[jax.ShapeDtypeStruct((MPAD, 8), jnp.float32)] * 3,
        scratch_shapes=[
            pltpu.VMEM((D, 128), jnp.float32),
            pltpu.VMEM((D, 128), jnp.float32),
        ],
    )(tu, ti, rel, mh, mb, mt)


def _sc_pick_body(yp_hbm, yh_hbm, yt_hbm, pidx_hbm, hidx_hbm, tidx_hbm,
                  outp, outh, outt, idx_v, val_v, sem):
    wid = lax.axis_index("s") * NUM_CORES + lax.axis_index("c")
    for y, idxs, out, cnt in ((yp_hbm, pidx_hbm, outp, GP_T),
                              (yh_hbm, hidx_hbm, outh, GH_T),
                              (yt_hbm, tidx_hbm, outt, GT_T)):
        base = wid * cnt
        pltpu.sync_copy(idxs.at[pl.ds(base, cnt)], idx_v.at[pl.ds(0, cnt)])

        def chunk(c0, y=y, cnt=cnt):
            vec = idx_v[pl.ds(c0 * KFIRE, KFIRE)]
            descs = []
            for i in range(KFIRE):
                descs.append(pltpu.async_copy(
                    y.at[pl.ds(vec[i], 1), :],
                    val_v.at[pl.ds(c0 * KFIRE + i, 1), :],
                    sem))
            for d in descs:
                d.wait()

        pl.loop(0, cnt // KFIRE)(chunk)
        pltpu.sync_copy(val_v.at[pl.ds(0, cnt), :], out.at[pl.ds(base, cnt), :])


def _run_sc_pick(yp, yh, yt, pidx, hidx, tidx):
    mesh = plsc.VectorSubcoreMesh(core_axis_name="c", subcore_axis_name="s")
    f = functools.partial(
        pl.kernel,
        mesh=mesh,
        out_type=[jax.ShapeDtypeStruct((E, 8), jnp.float32),
                  jax.ShapeDtypeStruct((E * NEG, 8), jnp.float32),
                  jax.ShapeDtypeStruct((E, 8), jnp.float32)],
        scratch_types=[
            pltpu.VMEM((GH_T,), jnp.int32),
            pltpu.VMEM((GH_T, 8), jnp.float32),
            pltpu.SemaphoreType.DMA,
        ],
    )(_sc_pick_body)
    return f(yp, yh, yt, pidx, hidx, tidx)


def kernel(emb_user, emb_item, relation_embedding, edge_index,
           edge_neg_head, edge_neg_tail):
    i32 = jnp.int32
    uh = edge_neg_head[:, 0].astype(i32)
    ib = edge_index[1].astype(i32)
    it = edge_neg_tail.reshape(-1).astype(i32)
    zeros_m = jnp.zeros((MPAD,), jnp.float32)

    mh, mb, mt = _run_sc_counts(uh, ib, it, zeros_m)

    yp8, yh8, yt8 = _run_tc_stream(
        emb_user.T, emb_item.T, relation_embedding[0],
        mh.reshape(1, MPAD), mb.reshape(1, MPAD), mt.reshape(1, MPAD))

    pidx = edge_index[0].astype(i32)
    hidx = edge_neg_head.reshape(-1).astype(i32)
    tidx = edge_neg_tail[:, 0].astype(i32)
    p8, h8, t8 = _run_sc_pick(yp8, yh8, yt8, pidx, hidx, tidx)

    return (p8[:, 0], h8[:, 0], jnp.repeat(t8[:, 0], NEG))


# speedup vs baseline: 2.8730x; 2.8730x over previous
import functools

import jax
import jax.numpy as jnp
from jax import lax
from jax.experimental import pallas as pl
from jax.experimental.pallas import tpu as pltpu
from jax.experimental.pallas import tpu_sc as plsc

N = 100000
D = 64
E = 1024
NEG = 4

NUM_CORES = 2
NUM_SUBCORES = 16
NW = NUM_CORES * NUM_SUBCORES

W = 25600
NB = 4
MPAD = W * NB
MSLICE = MPAD // NUM_SUBCORES

UH_T = E // NUM_SUBCORES
IB_T = E // NUM_SUBCORES
IT_T = E * NEG // NUM_SUBCORES

GP_T = E // NW
GH_T = E * NEG // NW
GT_T = E // NW
KFIRE = 16


def _sc_counts_body(uh_hbm, ib_hbm, it_hbm, z_hbm,
                    mh_out, mb_out, mt_out,
                    idx_s, idx_l, ones_v, vbuf,
                    acc_h, acc_b, acc_t, sem):
    cid = lax.axis_index("c")
    sid = lax.axis_index("s")
    for j in range(IT_T // 16):
        ones_v[pl.ds(j * 16, 16)] = jnp.full((16,), 1.0, jnp.float32)

    pltpu.sync_copy(z_hbm.at[pl.ds(sid * MSLICE, MSLICE)], vbuf)

    @pl.when(cid == 0)
    def _():
        pltpu.sync_copy(vbuf, acc_h.at[pl.ds(sid * MSLICE, MSLICE)])
        pltpu.sync_copy(vbuf, acc_b.at[pl.ds(sid * MSLICE, MSLICE)])

    @pl.when(cid == 1)
    def _():
        pltpu.sync_copy(vbuf, acc_t.at[pl.ds(sid * MSLICE, MSLICE)])

    plsc.subcore_barrier()

    @pl.when(cid == 0)
    def _():
        pltpu.sync_copy(uh_hbm.at[pl.ds(sid * UH_T, UH_T)], idx_s)
        pltpu.sync_copy(ones_v.at[pl.ds(0, UH_T)], acc_h.at[idx_s], add=True)
        pltpu.sync_copy(ib_hbm.at[pl.ds(sid * IB_T, IB_T)], idx_s)
        pltpu.sync_copy(ones_v.at[pl.ds(0, IB_T)], acc_b.at[idx_s], add=True)

    @pl.when(cid == 1)
    def _():
        pltpu.sync_copy(it_hbm.at[pl.ds(sid * IT_T, IT_T)], idx_l)
        pltpu.sync_copy(ones_v, acc_t.at[idx_l], add=True)

    plsc.subcore_barrier()

    @pl.when(cid == 0)
    def _():
        pltpu.sync_copy(acc_h.at[pl.ds(sid * MSLICE, MSLICE)], vbuf)
        pltpu.sync_copy(vbuf, mh_out.at[pl.ds(sid * MSLICE, MSLICE)])
        pltpu.sync_copy(acc_b.at[pl.ds(sid * MSLICE, MSLICE)], vbuf)
        pltpu.sync_copy(vbuf, mb_out.at[pl.ds(sid * MSLICE, MSLICE)])

    @pl.when(cid == 1)
    def _():
        pltpu.sync_copy(acc_t.at[pl.ds(sid * MSLICE, MSLICE)], vbuf)
        pltpu.sync_copy(vbuf, mt_out.at[pl.ds(sid * MSLICE, MSLICE)])


def _run_sc_counts(uh, ib, it, zeros_m):
    mesh = plsc.VectorSubcoreMesh(core_axis_name="c", subcore_axis_name="s")
    f = functools.partial(
        pl.kernel,
        mesh=mesh,
        out_type=[jax.ShapeDtypeStruct((MPAD,), jnp.float32)] * 3,
        scratch_types=[
            pltpu.VMEM((IB_T,), jnp.int32),
            pltpu.VMEM((IT_T,), jnp.int32),
            pltpu.VMEM((IT_T,), jnp.float32),
            pltpu.VMEM((MSLICE,), jnp.float32),
            pltpu.VMEM_SHARED((MPAD,), jnp.float32),
            pltpu.VMEM_SHARED((MPAD,), jnp.float32),
            pltpu.VMEM_SHARED((MPAD,), jnp.float32),
            pltpu.SemaphoreType.DMA,
        ],
    )(_sc_counts_body)
    return f(uh, ib, it, zeros_m)


def _tc_stream_body(tu_ref, ti_ref, k_ref, mh_ref, mb_ref, mt_ref,
                    yp_ref, yh_ref, yt_ref, s_ref, u_ref):
    g = pl.program_id(0)
    blk = lax.rem(g, NB)
    ds1 = (((1,), (1,)), ((), ()))
    cm = (((1,), (0,)), ((), ()))
    c0 = (((0,), (0,)), ((), ()))

    @pl.when(g < NB)
    def _phase_a():
        col = blk * W + lax.broadcasted_iota(jnp.int32, (1, W), 1)
        valid = col < N
        tu = jnp.where(valid, tu_ref[...], 0.0)
        ti = jnp.where(valid, ti_ref[...], 0.0)
        ph = lax.dot_general(tu, mh_ref[...], ds1,
                             preferred_element_type=jnp.float32)
        pb = lax.dot_general(ti, mb_ref[...], ds1,
                             preferred_element_type=jnp.float32)
        pt = lax.dot_general(ti, mt_ref[...], ds1,
                             preferred_element_type=jnp.float32)

        @pl.when(g == 0)
        def _():
            s_ref[:, 0:1] = ph
            s_ref[:, 1:2] = pb
            s_ref[:, 2:3] = pt

        @pl.when(g > 0)
        def _():
            s_ref[:, 0:1] = s_ref[:, 0:1] + ph
            s_ref[:, 1:2] = s_ref[:, 1:2] + pb
            s_ref[:, 2:3] = s_ref[:, 2:3] + pt

        zp = jnp.zeros((W,), jnp.float32)
        yp_ref[...] = zp
        yh_ref[...] = zp
        yt_ref[...] = zp

    @pl.when(g >= NB)
    def _phase_b():
        rel = k_ref[...]

        @pl.when(g == NB)
        def _():
            u_ref[:, 0:1] = lax.dot_general(
                rel, s_ref[:, 1:2], cm, preferred_element_type=jnp.float32)
            u_ref[:, 1:2] = float(NEG) * lax.dot_general(
                rel, s_ref[:, 0:1], cm, preferred_element_type=jnp.float32)
            u_ref[:, 2:3] = lax.dot_general(
                rel, s_ref[:, 2:3], cm, preferred_element_type=jnp.float32)

        tu = tu_ref[...]
        ti = ti_ref[...]
        ypc = lax.dot_general(u_ref[:, 0:1], tu, c0,
                              preferred_element_type=jnp.float32)
        yhc = lax.dot_general(u_ref[:, 1:2], tu, c0,
                              preferred_element_type=jnp.float32)
        ytc = lax.dot_general(u_ref[:, 2:3], ti, c0,
                              preferred_element_type=jnp.float32)
        yp_ref[...] = ypc[0]
        yh_ref[...] = yhc[0]
        yt_ref[...] = ytc[0]


def _run_tc_stream(tu, ti, rel, mh, mb, mt):
    node_blk = lambda g: (0, lax.rem(g, NB))
    return pl.pallas_call(
        _tc_stream_body,
        grid=(2 * NB,),
        in_specs=[
            pl.BlockSpec((D, W), node_blk),
            pl.BlockSpec((D, W), node_blk),
            pl.BlockSpec((D, D), lambda g: (0, 0)),
            pl.BlockSpec((1, W), node_blk),
            pl.BlockSpec((1, W), node_blk),
            pl.BlockSpec((1, W), node_blk),
        ],
        out_specs=[
            pl.BlockSpec((W,), lambda g: (lax.rem(g, NB),)),
            pl.BlockSpec((W,), lambda g: (lax.rem(g, NB),)),
            pl.BlockSpec((W,), lambda g: (lax.rem(g, NB),)),
        ],
        out_shape=[jax.ShapeDtypeStruct((MPAD,), jnp.float32)] * 3,
        scratch_shapes=[
            pltpu.VMEM((D, 128), jnp.float32),
            pltpu.VMEM((D, 128), jnp.float32),
        ],
    )(tu, ti, rel, mh, mb, mt)


def _sc_pick_body(yp_hbm, yh_hbm, yt_hbm, pidx_hbm, hidx_hbm, tidx_hbm,
                  outp, outh, outt, idx_v, win_v, out_v, sem):
    wid = lax.axis_index("s") * NUM_CORES + lax.axis_index("c")
    lane_ids = lax.iota(jnp.int32, 16)
    for y, idxs, out, cnt in ((yp_hbm, pidx_hbm, outp, GP_T),
                              (yh_hbm, hidx_hbm, outh, GH_T),
                              (yt_hbm, tidx_hbm, outt, GT_T)):
        base = wid * cnt
        pltpu.sync_copy(idxs.at[pl.ds(base, cnt)], idx_v.at[pl.ds(0, cnt)])

        def group(g, y=y):
            vec = idx_v[pl.ds(g * 16, 16)]
            descs = []
            for i in range(16):
                b8 = pl.multiple_of((vec[i] >> 3) * 8, 8)
                descs.append(pltpu.async_copy(
                    y.at[pl.ds(b8, 8)], win_v.at[pl.ds(i * 8, 8)], sem))
            for d in descs:
                d.wait()
            sel = plsc.load_gather(win_v, [lane_ids * 8 + (vec & 7)])
            out_v[pl.ds(g * 16, 16)] = sel

        pl.loop(0, cnt // 16)(group)
        pltpu.sync_copy(out_v.at[pl.ds(0, cnt)], out.at[pl.ds(base, cnt)])


def _run_sc_pick(yp, yh, yt, pidx, hidx, tidx):
    mesh = plsc.VectorSubcoreMesh(core_axis_name="c", subcore_axis_name="s")
    f = functools.partial(
        pl.kernel,
        mesh=mesh,
        out_type=[jax.ShapeDtypeStruct((E,), jnp.float32),
                  jax.ShapeDtypeStruct((E * NEG,), jnp.float32),
                  jax.ShapeDtypeStruct((E,), jnp.float32)],
        scratch_types=[
            pltpu.VMEM((GH_T,), jnp.int32),
            pltpu.VMEM((128,), jnp.float32),
            pltpu.VMEM((GH_T,), jnp.float32),
            pltpu.SemaphoreType.DMA,
        ],
        compiler_params=pltpu.CompilerParams(needs_layout_passes=False),
    )(_sc_pick_body)
    return f(yp, yh, yt, pidx, hidx, tidx)


def kernel(emb_user, emb_item, relation_embedding, edge_index,
           edge_neg_head, edge_neg_tail):
    i32 = jnp.int32
    uh = edge_neg_head[:, 0].astype(i32)
    ib = edge_index[1].astype(i32)
    it = edge_neg_tail.reshape(-1).astype(i32)
    zeros_m = jnp.zeros((MPAD,), jnp.float32)

    mh, mb, mt = _run_sc_counts(uh, ib, it, zeros_m)

    yp8, yh8, yt8 = _run_tc_stream(
        emb_user.T, emb_item.T, relation_embedding[0],
        mh.reshape(1, MPAD), mb.reshape(1, MPAD), mt.reshape(1, MPAD))

    pidx = edge_index[0].astype(i32)
    hidx = edge_neg_head.reshape(-1).astype(i32)
    tidx = edge_neg_tail[:, 0].astype(i32)
    score_pos, score_head, tail0 = _run_sc_pick(
        yp8, yh8, yt8, pidx, hidx, tidx)

    return (score_pos, score_head, jnp.repeat(tail0, NEG))


# speedup vs baseline: 3.0004x; 1.0443x over previous
import functools

import jax
import jax.numpy as jnp
from jax import lax
from jax.experimental import pallas as pl
from jax.experimental.pallas import tpu as pltpu
from jax.experimental.pallas import tpu_sc as plsc

N = 100000
D = 64
E = 1024
NEG = 4

NUM_CORES = 2
NUM_SUBCORES = 16
NW = NUM_CORES * NUM_SUBCORES

W = 25600
NB = 4
MPAD = W * NB
MSLICE = MPAD // NUM_SUBCORES

UH_T = E // NUM_SUBCORES
IB_T = E // NUM_SUBCORES
IT_T = E * NEG // NUM_SUBCORES

GP_T = E // NW
GH_T = E * NEG // NW
GT_T = E // NW
KFIRE = 16


def _sc_counts_body(uh_hbm, ib_hbm, it_hbm, z_hbm,
                    mh_out, mb_out, mt_out,
                    idx_s, idx_l, ones_v, vbuf,
                    acc_h, acc_b, acc_t, sem):
    cid = lax.axis_index("c")
    sid = lax.axis_index("s")
    for j in range(IT_T // 16):
        ones_v[pl.ds(j * 16, 16)] = jnp.full((16,), 1.0, jnp.float32)

    pltpu.sync_copy(z_hbm.at[pl.ds(sid * MSLICE, MSLICE)], vbuf)

    @pl.when(cid == 0)
    def _():
        pltpu.sync_copy(vbuf, acc_h.at[pl.ds(sid * MSLICE, MSLICE)])
        pltpu.sync_copy(vbuf, acc_b.at[pl.ds(sid * MSLICE, MSLICE)])

    @pl.when(cid == 1)
    def _():
        pltpu.sync_copy(vbuf, acc_t.at[pl.ds(sid * MSLICE, MSLICE)])

    plsc.subcore_barrier()

    @pl.when(cid == 0)
    def _():
        pltpu.sync_copy(uh_hbm.at[pl.ds(sid * UH_T, UH_T)], idx_s)
        pltpu.sync_copy(ones_v.at[pl.ds(0, UH_T)], acc_h.at[idx_s], add=True)
        pltpu.sync_copy(ib_hbm.at[pl.ds(sid * IB_T, IB_T)], idx_s)
        pltpu.sync_copy(ones_v.at[pl.ds(0, IB_T)], acc_b.at[idx_s], add=True)

    @pl.when(cid == 1)
    def _():
        pltpu.sync_copy(it_hbm.at[pl.ds(sid * IT_T, IT_T)], idx_l)
        pltpu.sync_copy(ones_v, acc_t.at[idx_l], add=True)

    plsc.subcore_barrier()

    @pl.when(cid == 0)
    def _():
        pltpu.sync_copy(acc_h.at[pl.ds(sid * MSLICE, MSLICE)], vbuf)
        pltpu.sync_copy(vbuf, mh_out.at[pl.ds(sid * MSLICE, MSLICE)])
        pltpu.sync_copy(acc_b.at[pl.ds(sid * MSLICE, MSLICE)], vbuf)
        pltpu.sync_copy(vbuf, mb_out.at[pl.ds(sid * MSLICE, MSLICE)])

    @pl.when(cid == 1)
    def _():
        pltpu.sync_copy(acc_t.at[pl.ds(sid * MSLICE, MSLICE)], vbuf)
        pltpu.sync_copy(vbuf, mt_out.at[pl.ds(sid * MSLICE, MSLICE)])


def _run_sc_counts(uh, ib, it, zeros_m):
    mesh = plsc.VectorSubcoreMesh(core_axis_name="c", subcore_axis_name="s")
    f = functools.partial(
        pl.kernel,
        mesh=mesh,
        out_type=[jax.ShapeDtypeStruct((MPAD,), jnp.float32)] * 3,
        scratch_types=[
            pltpu.VMEM((IB_T,), jnp.int32),
            pltpu.VMEM((IT_T,), jnp.int32),
            pltpu.VMEM((IT_T,), jnp.float32),
            pltpu.VMEM((MSLICE,), jnp.float32),
            pltpu.VMEM_SHARED((MPAD,), jnp.float32),
            pltpu.VMEM_SHARED((MPAD,), jnp.float32),
            pltpu.VMEM_SHARED((MPAD,), jnp.float32),
            pltpu.SemaphoreType.DMA,
        ],
    )(_sc_counts_body)
    return f(uh, ib, it, zeros_m)


def _tc_stream_body(tu_ref, ti_ref, k_ref, mh_ref, mb_ref, mt_ref,
                    yp_ref, yh_ref, yt_ref, s_ref, u_ref):
    g = pl.program_id(0)
    blk = lax.rem(g, NB)
    ds1 = (((1,), (1,)), ((), ()))
    cm = (((1,), (0,)), ((), ()))
    c0 = (((0,), (0,)), ((), ()))

    @pl.when(g < NB)
    def _phase_a():
        col = blk * W + lax.broadcasted_iota(jnp.int32, (1, W), 1)
        valid = col < N
        tu = jnp.where(valid, tu_ref[...], 0.0)
        ti = jnp.where(valid, ti_ref[...], 0.0)
        ph = lax.dot_general(tu, mh_ref[...], ds1,
                             preferred_element_type=jnp.float32)
        pb = lax.dot_general(ti, mb_ref[...], ds1,
                             preferred_element_type=jnp.float32)
        pt = lax.dot_general(ti, mt_ref[...], ds1,
                             preferred_element_type=jnp.float32)

        @pl.when(g == 0)
        def _():
            s_ref[:, 0:1] = ph
            s_ref[:, 1:2] = pb
            s_ref[:, 2:3] = pt

        @pl.when(g > 0)
        def _():
            s_ref[:, 0:1] = s_ref[:, 0:1] + ph
            s_ref[:, 1:2] = s_ref[:, 1:2] + pb
            s_ref[:, 2:3] = s_ref[:, 2:3] + pt

        zp = jnp.zeros((W,), jnp.float32)
        yp_ref[...] = zp
        yh_ref[...] = zp
        yt_ref[...] = zp

    @pl.when(g >= NB)
    def _phase_b():
        rel = k_ref[...]

        @pl.when(g == NB)
        def _():
            u_ref[:, 0:1] = lax.dot_general(
                rel, s_ref[:, 1:2], cm, preferred_element_type=jnp.float32)
            u_ref[:, 1:2] = float(NEG) * lax.dot_general(
                rel, s_ref[:, 0:1], cm, preferred_element_type=jnp.float32)
            u_ref[:, 2:3] = lax.dot_general(
                rel, s_ref[:, 2:3], cm, preferred_element_type=jnp.float32)

        tu = tu_ref[...]
        ti = ti_ref[...]
        ypc = lax.dot_general(u_ref[:, 0:1], tu, c0,
                              preferred_element_type=jnp.float32)
        yhc = lax.dot_general(u_ref[:, 1:2], tu, c0,
                              preferred_element_type=jnp.float32)
        ytc = lax.dot_general(u_ref[:, 2:3], ti, c0,
                              preferred_element_type=jnp.float32)
        yp_ref[...] = ypc[0]
        yh_ref[...] = yhc[0]
        yt_ref[...] = ytc[0]


def _run_tc_stream(tu, ti, rel, mh, mb, mt):
    node_blk = lambda g: (0, lax.rem(g, NB))
    return pl.pallas_call(
        _tc_stream_body,
        grid=(2 * NB,),
        in_specs=[
            pl.BlockSpec((D, W), node_blk),
            pl.BlockSpec((D, W), node_blk),
            pl.BlockSpec((D, D), lambda g: (0, 0)),
            pl.BlockSpec((1, W), node_blk),
            pl.BlockSpec((1, W), node_blk),
            pl.BlockSpec((1, W), node_blk),
        ],
        out_specs=[
            pl.BlockSpec((W,), lambda g: (lax.rem(g, NB),)),
            pl.BlockSpec((W,), lambda g: (lax.rem(g, NB),)),
            pl.BlockSpec((W,), lambda g: (lax.rem(g, NB),)),
        ],
        out_shape=[jax.ShapeDtypeStruct((MPAD,), jnp.float32)] * 3,
        scratch_shapes=[
            pltpu.VMEM((D, 128), jnp.float32),
            pltpu.VMEM((D, 128), jnp.float32),
        ],
    )(tu, ti, rel, mh, mb, mt)


def _sc_pick_body(yp_hbm, yh_hbm, yt_hbm, pidx_hbm, hidx_hbm, tidx_hbm,
                  outp, outh, outt, idx_v, win_v, out_v, sem):
    wid = lax.axis_index("s") * NUM_CORES + lax.axis_index("c")
    lane_ids = lax.iota(jnp.int32, 16)
    for y, idxs, out, cnt in ((yp_hbm, pidx_hbm, outp, GP_T),
                              (yh_hbm, hidx_hbm, outh, GH_T),
                              (yt_hbm, tidx_hbm, outt, GT_T)):
        base = wid * cnt
        pltpu.sync_copy(idxs.at[pl.ds(base, cnt)], idx_v.at[pl.ds(0, cnt)])

        ngroups = cnt // 16
        vecs, descs = [], []
        for g in range(ngroups):
            vec = idx_v[pl.ds(g * 16, 16)]
            vecs.append(vec)
            for i in range(16):
                b8 = pl.multiple_of((vec[i] >> 3) * 8, 8)
                descs.append(pltpu.async_copy(
                    y.at[pl.ds(b8, 8)],
                    win_v.at[pl.ds((g * 16 + i) * 8, 8)], sem))
        for d in descs:
            d.wait()
        for g in range(ngroups):
            sel = plsc.load_gather(
                win_v, [(g * 128) + lane_ids * 8 + (vecs[g] & 7)])
            out_v[pl.ds(g * 16, 16)] = sel
        pltpu.sync_copy(out_v.at[pl.ds(0, cnt)], out.at[pl.ds(base, cnt)])


def _run_sc_pick(yp, yh, yt, pidx, hidx, tidx):
    mesh = plsc.VectorSubcoreMesh(core_axis_name="c", subcore_axis_name="s")
    f = functools.partial(
        pl.kernel,
        mesh=mesh,
        out_type=[jax.ShapeDtypeStruct((E,), jnp.float32),
                  jax.ShapeDtypeStruct((E * NEG,), jnp.float32),
                  jax.ShapeDtypeStruct((E,), jnp.float32)],
        scratch_types=[
            pltpu.VMEM((GH_T,), jnp.int32),
            pltpu.VMEM((GH_T * 8,), jnp.float32),
            pltpu.VMEM((GH_T,), jnp.float32),
            pltpu.SemaphoreType.DMA,
        ],
        compiler_params=pltpu.CompilerParams(needs_layout_passes=False),
    )(_sc_pick_body)
    return f(yp, yh, yt, pidx, hidx, tidx)


def kernel(emb_user, emb_item, relation_embedding, edge_index,
           edge_neg_head, edge_neg_tail):
    i32 = jnp.int32
    uh = edge_neg_head[:, 0].astype(i32)
    ib = edge_index[1].astype(i32)
    it = edge_neg_tail.reshape(-1).astype(i32)
    zeros_m = jnp.zeros((MPAD,), jnp.float32)

    mh, mb, mt = _run_sc_counts(uh, ib, it, zeros_m)

    yp8, yh8, yt8 = _run_tc_stream(
        emb_user.T, emb_item.T, relation_embedding[0],
        mh.reshape(1, MPAD), mb.reshape(1, MPAD), mt.reshape(1, MPAD))

    pidx = edge_index[0].astype(i32)
    hidx = edge_neg_head.reshape(-1).astype(i32)
    tidx = edge_neg_tail[:, 0].astype(i32)
    score_pos, score_head, tail0 = _run_sc_pick(
        yp8, yh8, yt8, pidx, hidx, tidx)

    return (score_pos, score_head, jnp.repeat(tail0, NEG))
